# baseline (speedup 1.0000x reference)
import functools

import jax
import jax.numpy as jnp
from jax import lax
from jax.experimental import pallas as pl
from jax.experimental.pallas import tpu as pltpu
from jax.experimental.pallas import tpu_sc as plsc

N = 10000
E = 160000
F_IN = 256
H = 512
LANE = 128
C = H // LANE
NC = 2
NS = 16
G = 128
EPT = 10240
NG = EPT // G
NBUF = 2
NQ = 2
NGQ = NG // NQ
E_PAD = NS * EPT
ROWS_T = 624
LAST_T = N - (NS - 1) * ROWS_T
LAST_T16 = LAST_T + 16
NROW = N + 16
R = 1000
NEG = 0.01

_mesh = functools.partial(
    plsc.VectorSubcoreMesh,
    core_axis_name="c", subcore_axis_name="s", num_cores=NC, num_subcores=NS,
)


def _hop_rows(src_at, dst_at, nrows, bounce):
    nb = nrows // G
    rem = nrows - nb * G
    for b in range(nb):
        pltpu.sync_copy(src_at(b * G, G), bounce.at[pl.ds(0, G)])
        pltpu.sync_copy(bounce.at[pl.ds(0, G)], dst_at(b * G, G))
    if rem:
        pltpu.sync_copy(src_at(nb * G, rem), bounce.at[pl.ds(0, rem)])
        pltpu.sync_copy(bounce.at[pl.ds(0, rem)], dst_at(nb * G, rem))


def _deg_body(dstI, degp, dbuf, idx_d, ones_v, zv):
    cid = lax.axis_index("c")
    sid = lax.axis_index("s")
    for i in range(G // 16):
        ones_v[pl.ds(i * 16, 16)] = jnp.full((16,), 1.0, jnp.float32)
    for i in range(LAST_T16 // 16):
        zv[pl.ds(i * 16, 16)] = jnp.zeros((16,), jnp.float32)
    roff = pl.multiple_of(sid * ROWS_T, 8)

    @pl.when(sid < NS - 1)
    def _():
        pltpu.sync_copy(zv.at[pl.ds(0, ROWS_T)], dbuf.at[pl.ds(roff, ROWS_T)])

    @pl.when(sid == NS - 1)
    def _():
        pltpu.sync_copy(zv, dbuf.at[pl.ds(roff, LAST_T16)])

    plsc.subcore_barrier()

    def step(j, carry):
        pltpu.sync_copy(ones_v, dbuf.at[idx_d.at[j]], add=True)
        return carry

    for q in range(NQ // NC):
        pltpu.sync_copy(dstI.at[sid * NQ + cid * (NQ // NC) + q], idx_d)
        lax.fori_loop(0, NGQ, step, 0)
    plsc.subcore_barrier()

    doff = pl.multiple_of(cid * NROW + sid * ROWS_T, 8)

    @pl.when(sid < NS - 1)
    def _():
        pltpu.sync_copy(dbuf.at[pl.ds(roff, ROWS_T)], zv.at[pl.ds(0, ROWS_T)])
        pltpu.sync_copy(zv.at[pl.ds(0, ROWS_T)], degp.at[pl.ds(doff, ROWS_T)])

    @pl.when(sid == NS - 1)
    def _():
        pltpu.sync_copy(dbuf.at[pl.ds(roff, LAST_T)], zv.at[pl.ds(0, LAST_T)])
        pltpu.sync_copy(zv.at[pl.ds(0, LAST_T)], degp.at[pl.ds(doff, LAST_T)])


def _deg_call(dstI):
    return pl.kernel(
        _deg_body,
        out_type=jax.ShapeDtypeStruct((NC * NROW,), jnp.float32),
        mesh=_mesh(),
        scratch_types=[
            pltpu.VMEM_SHARED((NROW,), jnp.float32),
            pltpu.VMEM((NGQ, G), jnp.int32),
            pltpu.VMEM((G,), jnp.float32),
            pltpu.VMEM((LAST_T16,), jnp.float32),
        ],
    )(dstI)


def _scat_body(gflat, srcI, dstI, s_out, sbuf, idx_s, idx_d, rows, *sems):
    gsems = sems[:NBUF]
    ssems = sems[NBUF:]
    cid = lax.axis_index("c")
    sid = lax.axis_index("s")
    for k in range(C // NC):
        chunk = cid * (C // NC) + k
        base = chunk * N

        roff = pl.multiple_of(sid * ROWS_T, 8)
        goff = pl.multiple_of(base + sid * ROWS_T, 8)

        @pl.when(sid < NS - 1)
        def _():
            _hop_rows(lambda o, n: gflat.at[pl.ds(goff + o, n)],
                      lambda o, n: sbuf.at[pl.ds(roff + o, n)], ROWS_T, rows.at[0])

        @pl.when(sid == NS - 1)
        def _():
            _hop_rows(lambda o, n: gflat.at[pl.ds(goff + o, n)],
                      lambda o, n: sbuf.at[pl.ds(roff + o, n)], LAST_T, rows.at[0])

        plsc.subcore_barrier()

        NO = NGQ // NBUF
        for q in range(NQ):
            pltpu.sync_copy(srcI.at[(chunk * NS + sid) * NQ + q], idx_s)
            pltpu.sync_copy(dstI.at[sid * NQ + q], idx_d)

            for b in range(NBUF):
                pltpu.async_copy(gflat.at[idx_s.at[b]], rows.at[b], gsems[b])

            def step(o, carry):
                sdescs = []
                for b in range(NBUF):
                    j = o * NBUF + b
                    pltpu.make_async_copy(
                        gflat.at[pl.ds(0, G)], rows.at[b], gsems[b]).wait()
                    sdescs.append(pltpu.async_copy(
                        rows.at[b], sbuf.at[idx_d.at[j]], ssems[b], add=True))
                for b in range(NBUF):
                    sdescs[b].wait()

                    @pl.when(o < NO - 1)
                    def _():
                        jn = (o + 1) * NBUF + b
                        pltpu.async_copy(
                            gflat.at[idx_s.at[jn]], rows.at[b], gsems[b])
                return carry

            lax.fori_loop(0, NO, step, 0)
        plsc.subcore_barrier()

        @pl.when(sid < NS - 1)
        def _():
            _hop_rows(lambda o, n: sbuf.at[pl.ds(roff + o, n)],
                      lambda o, n: s_out.at[chunk, pl.ds(roff + o, n)], ROWS_T, rows.at[0])

        @pl.when(sid == NS - 1)
        def _():
            _hop_rows(lambda o, n: sbuf.at[pl.ds(roff + o, n)],
                      lambda o, n: s_out.at[chunk, pl.ds(roff + o, n)], LAST_T, rows.at[0])

        plsc.subcore_barrier()


def _scat_call(gflat, srcI, dstI):
    return pl.kernel(
        _scat_body,
        out_type=jax.ShapeDtypeStruct((C, N, LANE), jnp.float32),
        mesh=_mesh(),
        scratch_types=[
            pltpu.VMEM_SHARED((NROW, LANE), jnp.float32),
            pltpu.VMEM((NGQ, G), jnp.int32),
            pltpu.VMEM((NGQ, G), jnp.int32),
            pltpu.VMEM((NBUF, G, LANE), jnp.float32),
        ] + [pltpu.SemaphoreType.DMA] * (2 * NBUF),
    )(gflat, srcI, dstI)


def _dinv_body(degp_ref, dinv_ref):
    deg = degp_ref[0, :N] + degp_ref[1, :N] + 1.0
    dinv_ref[...] = lax.rsqrt(deg)[:, None]


def _dinv_call(degp):
    return pl.pallas_call(
        _dinv_body,
        grid=(1,),
        in_specs=[pl.BlockSpec((NC, NROW), lambda i: (0, 0))],
        out_specs=pl.BlockSpec((N, 1), lambda i: (0, 0)),
        out_shape=jax.ShapeDtypeStruct((N, 1), jnp.float32),
    )(degp)


def _mm1_body(x_ref, w_ref, dinv_ref, g_ref):
    dinv = dinv_ref[...]
    h = jnp.dot(x_ref[...], w_ref[...], preferred_element_type=jnp.float32)
    g = h * dinv
    for c in range(C):
        g_ref[c] = g[:, c * LANE:(c + 1) * LANE]


def _mid_body(s_ref, dinv_ref, b_ref, w_ref, g_ref):
    dinv = dinv_ref[...]
    acc = jnp.zeros((R, H), jnp.float32)
    for c in range(C):
        a = s_ref[c] * dinv + b_ref[0, c * LANE:(c + 1) * LANE][None, :]
        a = jnp.where(a > 0, a, NEG * a)
        acc = acc + jnp.dot(a, w_ref[c * LANE:(c + 1) * LANE, :],
                            preferred_element_type=jnp.float32)
    g = acc * dinv
    for c in range(C):
        g_ref[c] = g[:, c * LANE:(c + 1) * LANE]


def _fin_body(s_ref, dinv_ref, b_ref, wl_ref, bl_ref, o_ref):
    dinv = dinv_ref[...]
    acc = jnp.broadcast_to(bl_ref[0, :][None, :], (R, F_IN)).astype(jnp.float32)
    for c in range(C):
        a = s_ref[c] * dinv + b_ref[0, c * LANE:(c + 1) * LANE][None, :]
        acc = acc + jnp.dot(a, wl_ref[c * LANE:(c + 1) * LANE, :],
                            preferred_element_type=jnp.float32)
    o_ref[...] = acc


_g_spec = pl.BlockSpec((C, R, LANE), lambda i: (0, i, 0))
_dinv_spec = pl.BlockSpec((R, 1), lambda i: (i, 0))


def _mm1_call(x, W1, dinv):
    return pl.pallas_call(
        _mm1_body,
        grid=(N // R,),
        in_specs=[pl.BlockSpec((R, F_IN), lambda i: (i, 0)),
                  pl.BlockSpec((F_IN, H), lambda i: (0, 0)),
                  _dinv_spec],
        out_specs=_g_spec,
        out_shape=jax.ShapeDtypeStruct((C, N, LANE), jnp.float32),
    )(x, W1, dinv)


def _mid_call(s, dinv, b2d, W):
    return pl.pallas_call(
        _mid_body,
        grid=(N // R,),
        in_specs=[_g_spec,
                  _dinv_spec,
                  pl.BlockSpec((1, H), lambda i: (0, 0)),
                  pl.BlockSpec((H, H), lambda i: (0, 0))],
        out_specs=_g_spec,
        out_shape=jax.ShapeDtypeStruct((C, N, LANE), jnp.float32),
    )(s, dinv, b2d, W)


def _fin_call(s, dinv, b2d, Wl, bl2d):
    return pl.pallas_call(
        _fin_body,
        grid=(N // R,),
        in_specs=[_g_spec,
                  _dinv_spec,
                  pl.BlockSpec((1, H), lambda i: (0, 0)),
                  pl.BlockSpec((H, F_IN), lambda i: (0, 0)),
                  pl.BlockSpec((1, F_IN), lambda i: (0, 0))],
        out_specs=pl.BlockSpec((R, F_IN), lambda i: (i, 0)),
        out_shape=jax.ShapeDtypeStruct((N, F_IN), jnp.float32),
    )(s, dinv, b2d, Wl, bl2d)


def kernel(x, edge_index, W1, b1, W2, b2, W3, b3, Wl, bl):
    src = edge_index[0]
    dst = edge_index[1]
    pad = E_PAD - E
    src_p = jnp.concatenate([src, jnp.zeros((pad,), jnp.int32)])
    dst_p = jnp.concatenate([dst, jnp.full((pad,), N, jnp.int32)])
    dstI = dst_p.reshape(NS * NQ, NGQ, G)
    srcI = (src_p.reshape(1, NS * NQ, NGQ, G)
            + (jnp.arange(C, dtype=jnp.int32) * N)[:, None, None, None]
            ).reshape(C * NS * NQ, NGQ, G)
    degp = _deg_call(dstI)
    dinv = _dinv_call(degp.reshape(NC, NROW))

    g1 = _mm1_call(x, W1, dinv)
    s1 = _scat_call(g1.reshape(C * N, LANE), srcI, dstI)
    g2 = _mid_call(s1, dinv, b1.reshape(1, H), W2)
    s2 = _scat_call(g2.reshape(C * N, LANE), srcI, dstI)
    g3 = _mid_call(s2, dinv, b2.reshape(1, H), W3)
    s3 = _scat_call(g3.reshape(C * N, LANE), srcI, dstI)
    return _fin_call(s3, dinv, b3.reshape(1, H), Wl, bl.reshape(1, F_IN))


# speedup vs baseline: 1.0213x; 1.0213x over previous
import functools

import jax
import jax.numpy as jnp
from jax import lax
from jax.experimental import pallas as pl
from jax.experimental.pallas import tpu as pltpu
from jax.experimental.pallas import tpu_sc as plsc

N = 10000
E = 160000
F_IN = 256
H = 512
LANE = 128
C = H // LANE
NC = 2
NS = 16
G = 128
EPT = 10240
NG = EPT // G
NBUF = 2
NQ = 8
NGQ = NG // NQ
E_PAD = NS * EPT
ROWS_T = 624
LAST_T = N - (NS - 1) * ROWS_T
LAST_T16 = LAST_T + 16
NROW = N + 16
R = 1000
NEG = 0.01

_mesh = functools.partial(
    plsc.VectorSubcoreMesh,
    core_axis_name="c", subcore_axis_name="s", num_cores=NC, num_subcores=NS,
)


def _hop_rows(src_at, dst_at, nrows, bounce):
    nb = nrows // G
    rem = nrows - nb * G
    for b in range(nb):
        pltpu.sync_copy(src_at(b * G, G), bounce.at[pl.ds(0, G)])
        pltpu.sync_copy(bounce.at[pl.ds(0, G)], dst_at(b * G, G))
    if rem:
        pltpu.sync_copy(src_at(nb * G, rem), bounce.at[pl.ds(0, rem)])
        pltpu.sync_copy(bounce.at[pl.ds(0, rem)], dst_at(nb * G, rem))


def _deg_body(dstI, degp, dbuf, idx_d, ones_v, zv):
    cid = lax.axis_index("c")
    sid = lax.axis_index("s")
    for i in range(G // 16):
        ones_v[pl.ds(i * 16, 16)] = jnp.full((16,), 1.0, jnp.float32)
    for i in range(LAST_T16 // 16):
        zv[pl.ds(i * 16, 16)] = jnp.zeros((16,), jnp.float32)
    roff = pl.multiple_of(sid * ROWS_T, 8)

    @pl.when(sid < NS - 1)
    def _():
        pltpu.sync_copy(zv.at[pl.ds(0, ROWS_T)], dbuf.at[pl.ds(roff, ROWS_T)])

    @pl.when(sid == NS - 1)
    def _():
        pltpu.sync_copy(zv, dbuf.at[pl.ds(roff, LAST_T16)])

    plsc.subcore_barrier()

    def step(j, carry):
        pltpu.sync_copy(ones_v, dbuf.at[idx_d.at[j]], add=True)
        return carry

    for q in range(NQ // NC):
        pltpu.sync_copy(dstI.at[sid * NQ + cid * (NQ // NC) + q], idx_d)
        lax.fori_loop(0, NGQ, step, 0)
    plsc.subcore_barrier()

    doff = pl.multiple_of(cid * NROW + sid * ROWS_T, 8)

    @pl.when(sid < NS - 1)
    def _():
        pltpu.sync_copy(dbuf.at[pl.ds(roff, ROWS_T)], zv.at[pl.ds(0, ROWS_T)])
        pltpu.sync_copy(zv.at[pl.ds(0, ROWS_T)], degp.at[pl.ds(doff, ROWS_T)])

    @pl.when(sid == NS - 1)
    def _():
        pltpu.sync_copy(dbuf.at[pl.ds(roff, LAST_T)], zv.at[pl.ds(0, LAST_T)])
        pltpu.sync_copy(zv.at[pl.ds(0, LAST_T)], degp.at[pl.ds(doff, LAST_T)])


def _deg_call(dstI):
    return pl.kernel(
        _deg_body,
        out_type=jax.ShapeDtypeStruct((NC * NROW,), jnp.float32),
        mesh=_mesh(),
        scratch_types=[
            pltpu.VMEM_SHARED((NROW,), jnp.float32),
            pltpu.VMEM((NGQ, G), jnp.int32),
            pltpu.VMEM((G,), jnp.float32),
            pltpu.VMEM((LAST_T16,), jnp.float32),
        ],
    )(dstI)


def _scat_body(gflat, srcI, dstI, s_out, sbuf, idx_s, idx_d, rows, *sems):
    gsems = sems[:NBUF]
    ssems = sems[NBUF:]
    cid = lax.axis_index("c")
    sid = lax.axis_index("s")
    for k in range(C // NC):
        chunk = cid * (C // NC) + k
        base = chunk * N

        roff = pl.multiple_of(sid * ROWS_T, 8)
        goff = pl.multiple_of(base + sid * ROWS_T, 8)

        @pl.when(sid < NS - 1)
        def _():
            _hop_rows(lambda o, n: gflat.at[pl.ds(goff + o, n)],
                      lambda o, n: sbuf.at[pl.ds(roff + o, n)], ROWS_T, rows.at[0])

        @pl.when(sid == NS - 1)
        def _():
            _hop_rows(lambda o, n: gflat.at[pl.ds(goff + o, n)],
                      lambda o, n: sbuf.at[pl.ds(roff + o, n)], LAST_T, rows.at[0])

        plsc.subcore_barrier()

        NO = NGQ // NBUF
        for q in range(NQ):
            pltpu.sync_copy(srcI.at[(chunk * NS + sid) * NQ + q], idx_s)
            pltpu.sync_copy(dstI.at[sid * NQ + q], idx_d)

            for b in range(NBUF):
                pltpu.async_copy(gflat.at[idx_s.at[b]], rows.at[b], gsems[b])

            def step(o, carry):
                sdescs = []
                for b in range(NBUF):
                    j = o * NBUF + b
                    pltpu.make_async_copy(
                        gflat.at[pl.ds(0, G)], rows.at[b], gsems[b]).wait()
                    sdescs.append(pltpu.async_copy(
                        rows.at[b], sbuf.at[idx_d.at[j]], ssems[b], add=True))
                for b in range(NBUF):
                    sdescs[b].wait()

                    @pl.when(o < NO - 1)
                    def _():
                        jn = (o + 1) * NBUF + b
                        pltpu.async_copy(
                            gflat.at[idx_s.at[jn]], rows.at[b], gsems[b])
                return carry

            lax.fori_loop(0, NO, step, 0)
        plsc.subcore_barrier()

        @pl.when(sid < NS - 1)
        def _():
            _hop_rows(lambda o, n: sbuf.at[pl.ds(roff + o, n)],
                      lambda o, n: s_out.at[chunk, pl.ds(roff + o, n)], ROWS_T, rows.at[0])

        @pl.when(sid == NS - 1)
        def _():
            _hop_rows(lambda o, n: sbuf.at[pl.ds(roff + o, n)],
                      lambda o, n: s_out.at[chunk, pl.ds(roff + o, n)], LAST_T, rows.at[0])

        plsc.subcore_barrier()


def _scat_call(gflat, srcI, dstI):
    return pl.kernel(
        _scat_body,
        out_type=jax.ShapeDtypeStruct((C, N, LANE), jnp.float32),
        mesh=_mesh(),
        scratch_types=[
            pltpu.VMEM_SHARED((NROW, LANE), jnp.float32),
            pltpu.VMEM((NGQ, G), jnp.int32),
            pltpu.VMEM((NGQ, G), jnp.int32),
            pltpu.VMEM((NBUF, G, LANE), jnp.float32),
        ] + [pltpu.SemaphoreType.DMA] * (2 * NBUF),
    )(gflat, srcI, dstI)


def _dinv_body(degp_ref, dinv_ref):
    deg = degp_ref[0, :N] + degp_ref[1, :N] + 1.0
    dinv_ref[...] = lax.rsqrt(deg)[:, None]


def _dinv_call(degp):
    return pl.pallas_call(
        _dinv_body,
        grid=(1,),
        in_specs=[pl.BlockSpec((NC, NROW), lambda i: (0, 0))],
        out_specs=pl.BlockSpec((N, 1), lambda i: (0, 0)),
        out_shape=jax.ShapeDtypeStruct((N, 1), jnp.float32),
    )(degp)


def _mm1_body(x_ref, w_ref, dinv_ref, g_ref):
    dinv = dinv_ref[...]
    h = jnp.dot(x_ref[...], w_ref[...], preferred_element_type=jnp.float32)
    g = h * dinv
    for c in range(C):
        g_ref[c] = g[:, c * LANE:(c + 1) * LANE]


def _mid_body(s_ref, dinv_ref, b_ref, w_ref, g_ref):
    dinv = dinv_ref[...]
    acc = jnp.zeros((R, H), jnp.float32)
    for c in range(C):
        a = s_ref[c] * dinv + b_ref[0, c * LANE:(c + 1) * LANE][None, :]
        a = jnp.where(a > 0, a, NEG * a)
        acc = acc + jnp.dot(a, w_ref[c * LANE:(c + 1) * LANE, :],
                            preferred_element_type=jnp.float32)
    g = acc * dinv
    for c in range(C):
        g_ref[c] = g[:, c * LANE:(c + 1) * LANE]


def _fin_body(s_ref, dinv_ref, b_ref, wl_ref, bl_ref, o_ref):
    dinv = dinv_ref[...]
    acc = jnp.broadcast_to(bl_ref[0, :][None, :], (R, F_IN)).astype(jnp.float32)
    for c in range(C):
        a = s_ref[c] * dinv + b_ref[0, c * LANE:(c + 1) * LANE][None, :]
        acc = acc + jnp.dot(a, wl_ref[c * LANE:(c + 1) * LANE, :],
                            preferred_element_type=jnp.float32)
    o_ref[...] = acc


_g_spec = pl.BlockSpec((C, R, LANE), lambda i: (0, i, 0))
_dinv_spec = pl.BlockSpec((R, 1), lambda i: (i, 0))


def _mm1_call(x, W1, dinv):
    return pl.pallas_call(
        _mm1_body,
        grid=(N // R,),
        in_specs=[pl.BlockSpec((R, F_IN), lambda i: (i, 0)),
                  pl.BlockSpec((F_IN, H), lambda i: (0, 0)),
                  _dinv_spec],
        out_specs=_g_spec,
        out_shape=jax.ShapeDtypeStruct((C, N, LANE), jnp.float32),
    )(x, W1, dinv)


def _mid_call(s, dinv, b2d, W):
    return pl.pallas_call(
        _mid_body,
        grid=(N // R,),
        in_specs=[_g_spec,
                  _dinv_spec,
                  pl.BlockSpec((1, H), lambda i: (0, 0)),
                  pl.BlockSpec((H, H), lambda i: (0, 0))],
        out_specs=_g_spec,
        out_shape=jax.ShapeDtypeStruct((C, N, LANE), jnp.float32),
    )(s, dinv, b2d, W)


def _fin_call(s, dinv, b2d, Wl, bl2d):
    return pl.pallas_call(
        _fin_body,
        grid=(N // R,),
        in_specs=[_g_spec,
                  _dinv_spec,
                  pl.BlockSpec((1, H), lambda i: (0, 0)),
                  pl.BlockSpec((H, F_IN), lambda i: (0, 0)),
                  pl.BlockSpec((1, F_IN), lambda i: (0, 0))],
        out_specs=pl.BlockSpec((R, F_IN), lambda i: (i, 0)),
        out_shape=jax.ShapeDtypeStruct((N, F_IN), jnp.float32),
    )(s, dinv, b2d, Wl, bl2d)


def kernel(x, edge_index, W1, b1, W2, b2, W3, b3, Wl, bl):
    src = edge_index[0]
    dst = edge_index[1]
    pad = E_PAD - E
    src_p = jnp.concatenate([src, jnp.zeros((pad,), jnp.int32)])
    dst_p = jnp.concatenate([dst, jnp.full((pad,), N, jnp.int32)])
    dstI = dst_p.reshape(NS * NQ, NGQ, G)
    srcI = (src_p.reshape(1, NS * NQ, NGQ, G)
            + (jnp.arange(C, dtype=jnp.int32) * N)[:, None, None, None]
            ).reshape(C * NS * NQ, NGQ, G)
    degp = _deg_call(dstI)
    dinv = _dinv_call(degp.reshape(NC, NROW))

    g1 = _mm1_call(x, W1, dinv)
    s1 = _scat_call(g1.reshape(C * N, LANE), srcI, dstI)
    g2 = _mid_call(s1, dinv, b1.reshape(1, H), W2)
    s2 = _scat_call(g2.reshape(C * N, LANE), srcI, dstI)
    g3 = _mid_call(s2, dinv, b2.reshape(1, H), W3)
    s3 = _scat_call(g3.reshape(C * N, LANE), srcI, dstI)
    return _fin_call(s3, dinv, b3.reshape(1, H), Wl, bl.reshape(1, F_IN))


# speedup vs baseline: 1.0526x; 1.0306x over previous
import functools

import jax
import jax.numpy as jnp
from jax import lax
from jax.experimental import pallas as pl
from jax.experimental.pallas import tpu as pltpu
from jax.experimental.pallas import tpu_sc as plsc

N = 10000
E = 160000
F_IN = 256
H = 512
LANE = 128
C = H // LANE
NC = 2
NS = 16
G = 128
EPT = 10240
NG = EPT // G
NBUF = 2
NQ = 4
NGQ = NG // NQ
E_PAD = NS * EPT
ROWS_T = 624
LAST_T = N - (NS - 1) * ROWS_T
LAST_T16 = LAST_T + 16
NROW = N + 16
R = 1000
NEG = 0.01

_mesh = functools.partial(
    plsc.VectorSubcoreMesh,
    core_axis_name="c", subcore_axis_name="s", num_cores=NC, num_subcores=NS,
)


def _hop_rows(src_at, dst_at, nrows, bounce):
    nb = nrows // G
    rem = nrows - nb * G
    for b in range(nb):
        pltpu.sync_copy(src_at(b * G, G), bounce.at[pl.ds(0, G)])
        pltpu.sync_copy(bounce.at[pl.ds(0, G)], dst_at(b * G, G))
    if rem:
        pltpu.sync_copy(src_at(nb * G, rem), bounce.at[pl.ds(0, rem)])
        pltpu.sync_copy(bounce.at[pl.ds(0, rem)], dst_at(nb * G, rem))


def _deg_body(dstI, degp, dbuf, idx_d, ones_v, zv):
    cid = lax.axis_index("c")
    sid = lax.axis_index("s")
    for i in range(G // 16):
        ones_v[pl.ds(i * 16, 16)] = jnp.full((16,), 1.0, jnp.float32)
    for i in range(LAST_T16 // 16):
        zv[pl.ds(i * 16, 16)] = jnp.zeros((16,), jnp.float32)
    roff = pl.multiple_of(sid * ROWS_T, 8)

    @pl.when(sid < NS - 1)
    def _():
        pltpu.sync_copy(zv.at[pl.ds(0, ROWS_T)], dbuf.at[pl.ds(roff, ROWS_T)])

    @pl.when(sid == NS - 1)
    def _():
        pltpu.sync_copy(zv, dbuf.at[pl.ds(roff, LAST_T16)])

    plsc.subcore_barrier()

    def step(j, carry):
        pltpu.sync_copy(ones_v, dbuf.at[idx_d.at[j]], add=True)
        return carry

    for q in range(NQ // NC):
        pltpu.sync_copy(dstI.at[sid * NQ + cid * (NQ // NC) + q], idx_d)
        lax.fori_loop(0, NGQ, step, 0)
    plsc.subcore_barrier()

    doff = pl.multiple_of(cid * NROW + sid * ROWS_T, 8)

    @pl.when(sid < NS - 1)
    def _():
        pltpu.sync_copy(dbuf.at[pl.ds(roff, ROWS_T)], zv.at[pl.ds(0, ROWS_T)])
        pltpu.sync_copy(zv.at[pl.ds(0, ROWS_T)], degp.at[pl.ds(doff, ROWS_T)])

    @pl.when(sid == NS - 1)
    def _():
        pltpu.sync_copy(dbuf.at[pl.ds(roff, LAST_T)], zv.at[pl.ds(0, LAST_T)])
        pltpu.sync_copy(zv.at[pl.ds(0, LAST_T)], degp.at[pl.ds(doff, LAST_T)])


def _deg_call(dstI):
    return pl.kernel(
        _deg_body,
        out_type=jax.ShapeDtypeStruct((NC * NROW,), jnp.float32),
        mesh=_mesh(),
        scratch_types=[
            pltpu.VMEM_SHARED((NROW,), jnp.float32),
            pltpu.VMEM((NGQ, G), jnp.int32),
            pltpu.VMEM((G,), jnp.float32),
            pltpu.VMEM((LAST_T16,), jnp.float32),
        ],
    )(dstI)


def _scat_body(gflat, srcI, dstI, s_out, sbuf, idx_s, idx_d, rows, *sems):
    gsems = sems[:NBUF]
    ssems = sems[NBUF:]
    cid = lax.axis_index("c")
    sid = lax.axis_index("s")
    for k in range(C // NC):
        chunk = cid * (C // NC) + k
        base = chunk * N

        roff = pl.multiple_of(sid * ROWS_T, 8)
        goff = pl.multiple_of(base + sid * ROWS_T, 8)

        @pl.when(sid < NS - 1)
        def _():
            _hop_rows(lambda o, n: gflat.at[pl.ds(goff + o, n)],
                      lambda o, n: sbuf.at[pl.ds(roff + o, n)], ROWS_T, rows.at[0])

        @pl.when(sid == NS - 1)
        def _():
            _hop_rows(lambda o, n: gflat.at[pl.ds(goff + o, n)],
                      lambda o, n: sbuf.at[pl.ds(roff + o, n)], LAST_T, rows.at[0])

        plsc.subcore_barrier()

        NO = NGQ // NBUF
        for q in range(NQ):
            pltpu.sync_copy(srcI.at[(chunk * NS + sid) * NQ + q], idx_s)
            pltpu.sync_copy(dstI.at[sid * NQ + q], idx_d)

            for b in range(NBUF):
                pltpu.async_copy(gflat.at[idx_s.at[b]], rows.at[b], gsems[b])

            def step(o, carry):
                sdescs = []
                for b in range(NBUF):
                    j = o * NBUF + b
                    pltpu.make_async_copy(
                        gflat.at[pl.ds(0, G)], rows.at[b], gsems[b]).wait()
                    sdescs.append(pltpu.async_copy(
                        rows.at[b], sbuf.at[idx_d.at[j]], ssems[b], add=True))
                for b in range(NBUF):
                    sdescs[b].wait()

                    @pl.when(o < NO - 1)
                    def _():
                        jn = (o + 1) * NBUF + b
                        pltpu.async_copy(
                            gflat.at[idx_s.at[jn]], rows.at[b], gsems[b])
                return carry

            lax.fori_loop(0, NO, step, 0)
        plsc.subcore_barrier()

        @pl.when(sid < NS - 1)
        def _():
            _hop_rows(lambda o, n: sbuf.at[pl.ds(roff + o, n)],
                      lambda o, n: s_out.at[chunk, pl.ds(roff + o, n)], ROWS_T, rows.at[0])

        @pl.when(sid == NS - 1)
        def _():
            _hop_rows(lambda o, n: sbuf.at[pl.ds(roff + o, n)],
                      lambda o, n: s_out.at[chunk, pl.ds(roff + o, n)], LAST_T, rows.at[0])

        plsc.subcore_barrier()


def _scat_call(gflat, srcI, dstI):
    return pl.kernel(
        _scat_body,
        out_type=jax.ShapeDtypeStruct((C, N, LANE), jnp.float32),
        mesh=_mesh(),
        scratch_types=[
            pltpu.VMEM_SHARED((NROW, LANE), jnp.float32),
            pltpu.VMEM((NGQ, G), jnp.int32),
            pltpu.VMEM((NGQ, G), jnp.int32),
            pltpu.VMEM((NBUF, G, LANE), jnp.float32),
        ] + [pltpu.SemaphoreType.DMA] * (2 * NBUF),
    )(gflat, srcI, dstI)


def _dinv_body(degp_ref, dinv_ref):
    deg = degp_ref[0, :N] + degp_ref[1, :N] + 1.0
    dinv_ref[...] = lax.rsqrt(deg)[:, None]


def _dinv_call(degp):
    return pl.pallas_call(
        _dinv_body,
        grid=(1,),
        in_specs=[pl.BlockSpec((NC, NROW), lambda i: (0, 0))],
        out_specs=pl.BlockSpec((N, 1), lambda i: (0, 0)),
        out_shape=jax.ShapeDtypeStruct((N, 1), jnp.float32),
    )(degp)


def _mm1_body(x_ref, w_ref, dinv_ref, g_ref):
    dinv = dinv_ref[...]
    h = jnp.dot(x_ref[...].astype(jnp.bfloat16), w_ref[...].astype(jnp.bfloat16),
                preferred_element_type=jnp.float32)
    g = h * dinv
    for c in range(C):
        g_ref[c] = g[:, c * LANE:(c + 1) * LANE]


def _mid_body(s_ref, dinv_ref, b_ref, w_ref, g_ref):
    dinv = dinv_ref[...]
    acc = jnp.zeros((R, H), jnp.float32)
    for c in range(C):
        a = s_ref[c] * dinv + b_ref[0, c * LANE:(c + 1) * LANE][None, :]
        a = jnp.where(a > 0, a, NEG * a)
        acc = acc + jnp.dot(a.astype(jnp.bfloat16),
                            w_ref[c * LANE:(c + 1) * LANE, :].astype(jnp.bfloat16),
                            preferred_element_type=jnp.float32)
    g = acc * dinv
    for c in range(C):
        g_ref[c] = g[:, c * LANE:(c + 1) * LANE]


def _fin_body(s_ref, dinv_ref, b_ref, wl_ref, bl_ref, o_ref):
    dinv = dinv_ref[...]
    acc = jnp.broadcast_to(bl_ref[0, :][None, :], (R, F_IN)).astype(jnp.float32)
    for c in range(C):
        a = s_ref[c] * dinv + b_ref[0, c * LANE:(c + 1) * LANE][None, :]
        acc = acc + jnp.dot(a.astype(jnp.bfloat16),
                            wl_ref[c * LANE:(c + 1) * LANE, :].astype(jnp.bfloat16),
                            preferred_element_type=jnp.float32)
    o_ref[...] = acc


_g_spec = pl.BlockSpec((C, R, LANE), lambda i: (0, i, 0))
_dinv_spec = pl.BlockSpec((R, 1), lambda i: (i, 0))


def _mm1_call(x, W1, dinv):
    return pl.pallas_call(
        _mm1_body,
        grid=(N // R,),
        in_specs=[pl.BlockSpec((R, F_IN), lambda i: (i, 0)),
                  pl.BlockSpec((F_IN, H), lambda i: (0, 0)),
                  _dinv_spec],
        out_specs=_g_spec,
        out_shape=jax.ShapeDtypeStruct((C, N, LANE), jnp.float32),
    )(x, W1, dinv)


def _mid_call(s, dinv, b2d, W):
    return pl.pallas_call(
        _mid_body,
        grid=(N // R,),
        in_specs=[_g_spec,
                  _dinv_spec,
                  pl.BlockSpec((1, H), lambda i: (0, 0)),
                  pl.BlockSpec((H, H), lambda i: (0, 0))],
        out_specs=_g_spec,
        out_shape=jax.ShapeDtypeStruct((C, N, LANE), jnp.float32),
    )(s, dinv, b2d, W)


def _fin_call(s, dinv, b2d, Wl, bl2d):
    return pl.pallas_call(
        _fin_body,
        grid=(N // R,),
        in_specs=[_g_spec,
                  _dinv_spec,
                  pl.BlockSpec((1, H), lambda i: (0, 0)),
                  pl.BlockSpec((H, F_IN), lambda i: (0, 0)),
                  pl.BlockSpec((1, F_IN), lambda i: (0, 0))],
        out_specs=pl.BlockSpec((R, F_IN), lambda i: (i, 0)),
        out_shape=jax.ShapeDtypeStruct((N, F_IN), jnp.float32),
    )(s, dinv, b2d, Wl, bl2d)


def kernel(x, edge_index, W1, b1, W2, b2, W3, b3, Wl, bl):
    src = edge_index[0]
    dst = edge_index[1]
    pad = E_PAD - E
    src_p = jnp.concatenate([src, jnp.zeros((pad,), jnp.int32)])
    dst_p = jnp.concatenate([dst, jnp.full((pad,), N, jnp.int32)])
    dstI = dst_p.reshape(NS * NQ, NGQ, G)
    srcI = (src_p.reshape(1, NS * NQ, NGQ, G)
            + (jnp.arange(C, dtype=jnp.int32) * N)[:, None, None, None]
            ).reshape(C * NS * NQ, NGQ, G)
    degp = _deg_call(dstI)
    dinv = _dinv_call(degp.reshape(NC, NROW))

    g1 = _mm1_call(x, W1, dinv)
    s1 = _scat_call(g1.reshape(C * N, LANE), srcI, dstI)
    g2 = _mid_call(s1, dinv, b1.reshape(1, H), W2)
    s2 = _scat_call(g2.reshape(C * N, LANE), srcI, dstI)
    g3 = _mid_call(s2, dinv, b2.reshape(1, H), W3)
    s3 = _scat_call(g3.reshape(C * N, LANE), srcI, dstI)
    return _fin_call(s3, dinv, b3.reshape(1, H), Wl, bl.reshape(1, F_IN))


# speedup vs baseline: 1.0581x; 1.0053x over previous
import functools

import jax
import jax.numpy as jnp
from jax import lax
from jax.experimental import pallas as pl
from jax.experimental.pallas import tpu as pltpu
from jax.experimental.pallas import tpu_sc as plsc

N = 10000
E = 160000
F_IN = 256
H = 512
LANE = 128
C = H // LANE
NC = 2
NS = 16
G = 128
EPT = 10240
NG = EPT // G
NBUF = 2
NQ = 4
NGQ = NG // NQ
E_PAD = NS * EPT
ROWS_T = 624
LAST_T = N - (NS - 1) * ROWS_T
LAST_T16 = LAST_T + 16
NROW = N + 16
R = 1000
NEG = 0.01

_mesh = functools.partial(
    plsc.VectorSubcoreMesh,
    core_axis_name="c", subcore_axis_name="s", num_cores=NC, num_subcores=NS,
)


def _hop_rows(src_at, dst_at, nrows, bounce):
    nb = nrows // G
    rem = nrows - nb * G
    for b in range(nb):
        pltpu.sync_copy(src_at(b * G, G), bounce.at[pl.ds(0, G)])
        pltpu.sync_copy(bounce.at[pl.ds(0, G)], dst_at(b * G, G))
    if rem:
        pltpu.sync_copy(src_at(nb * G, rem), bounce.at[pl.ds(0, rem)])
        pltpu.sync_copy(bounce.at[pl.ds(0, rem)], dst_at(nb * G, rem))


def _deg_body(dstI, degp, dbuf, idx_d, ones_v, zv):
    cid = lax.axis_index("c")
    sid = lax.axis_index("s")
    for i in range(G // 16):
        ones_v[pl.ds(i * 16, 16)] = jnp.full((16,), 1.0, jnp.float32)
    for i in range(LAST_T16 // 16):
        zv[pl.ds(i * 16, 16)] = jnp.zeros((16,), jnp.float32)
    roff = pl.multiple_of(sid * ROWS_T, 8)

    @pl.when(sid < NS - 1)
    def _():
        pltpu.sync_copy(zv.at[pl.ds(0, ROWS_T)], dbuf.at[pl.ds(roff, ROWS_T)])

    @pl.when(sid == NS - 1)
    def _():
        pltpu.sync_copy(zv, dbuf.at[pl.ds(roff, LAST_T16)])

    plsc.subcore_barrier()

    def step(j, carry):
        pltpu.sync_copy(ones_v, dbuf.at[idx_d.at[j]], add=True)
        return carry

    for q in range(NQ // NC):
        pltpu.sync_copy(dstI.at[sid * NQ + cid * (NQ // NC) + q], idx_d)
        lax.fori_loop(0, NGQ, step, 0)
    plsc.subcore_barrier()

    doff = pl.multiple_of(cid * NROW + sid * ROWS_T, 8)

    @pl.when(sid < NS - 1)
    def _():
        pltpu.sync_copy(dbuf.at[pl.ds(roff, ROWS_T)], zv.at[pl.ds(0, ROWS_T)])
        pltpu.sync_copy(zv.at[pl.ds(0, ROWS_T)], degp.at[pl.ds(doff, ROWS_T)])

    @pl.when(sid == NS - 1)
    def _():
        pltpu.sync_copy(dbuf.at[pl.ds(roff, LAST_T)], zv.at[pl.ds(0, LAST_T)])
        pltpu.sync_copy(zv.at[pl.ds(0, LAST_T)], degp.at[pl.ds(doff, LAST_T)])


def _deg_call(dstI):
    return pl.kernel(
        _deg_body,
        out_type=jax.ShapeDtypeStruct((NC * NROW,), jnp.float32),
        mesh=_mesh(),
        scratch_types=[
            pltpu.VMEM_SHARED((NROW,), jnp.float32),
            pltpu.VMEM((NGQ, G), jnp.int32),
            pltpu.VMEM((G,), jnp.float32),
            pltpu.VMEM((LAST_T16,), jnp.float32),
        ],
    )(dstI)


def _scat_body(gflat, srcI, dstI, s_out, sbuf, idx_s, idx_d, rows, *sems):
    gsems = sems[:NBUF]
    ssems = sems[NBUF:]
    cid = lax.axis_index("c")
    sid = lax.axis_index("s")
    for k in range(C // NC):
        chunk = cid * (C // NC) + k
        base = chunk * N

        roff = pl.multiple_of(sid * ROWS_T, 8)
        goff = pl.multiple_of(base + sid * ROWS_T, 8)

        @pl.when(sid < NS - 1)
        def _():
            _hop_rows(lambda o, n: gflat.at[pl.ds(goff + o, n)],
                      lambda o, n: sbuf.at[pl.ds(roff + o, n)], ROWS_T, rows.at[0])

        @pl.when(sid == NS - 1)
        def _():
            _hop_rows(lambda o, n: gflat.at[pl.ds(goff + o, n)],
                      lambda o, n: sbuf.at[pl.ds(roff + o, n)], LAST_T, rows.at[0])

        plsc.subcore_barrier()

        NO = NGQ // NBUF
        for q in range(NQ):
            pltpu.sync_copy(srcI.at[(chunk * NS + sid) * NQ + q], idx_s)
            pltpu.sync_copy(dstI.at[sid * NQ + q], idx_d)

            for b in range(NBUF):
                pltpu.async_copy(gflat.at[idx_s.at[b]], rows.at[b], gsems[b])

            def step(o, carry):
                sdescs = []
                for b in range(NBUF):
                    j = o * NBUF + b
                    pltpu.make_async_copy(
                        gflat.at[pl.ds(0, G)], rows.at[b], gsems[b]).wait()
                    sdescs.append(pltpu.async_copy(
                        rows.at[b], sbuf.at[idx_d.at[j]], ssems[b], add=True))
                for b in range(NBUF):
                    sdescs[b].wait()

                    @pl.when(o < NO - 1)
                    def _():
                        jn = (o + 1) * NBUF + b
                        pltpu.async_copy(
                            gflat.at[idx_s.at[jn]], rows.at[b], gsems[b])
                return carry

            lax.fori_loop(0, NO, step, 0)
        plsc.subcore_barrier()

        @pl.when(sid < NS - 1)
        def _():
            _hop_rows(lambda o, n: sbuf.at[pl.ds(roff + o, n)],
                      lambda o, n: s_out.at[chunk, pl.ds(roff + o, n)], ROWS_T, rows.at[0])

        @pl.when(sid == NS - 1)
        def _():
            _hop_rows(lambda o, n: sbuf.at[pl.ds(roff + o, n)],
                      lambda o, n: s_out.at[chunk, pl.ds(roff + o, n)], LAST_T, rows.at[0])

        plsc.subcore_barrier()


def _scat_call(gflat, srcI, dstI):
    return pl.kernel(
        _scat_body,
        out_type=jax.ShapeDtypeStruct((C, N, LANE), jnp.float32),
        mesh=_mesh(),
        scratch_types=[
            pltpu.VMEM_SHARED((NROW, LANE), jnp.float32),
            pltpu.VMEM((NGQ, G), jnp.int32),
            pltpu.VMEM((NGQ, G), jnp.int32),
            pltpu.VMEM((NBUF, G, LANE), jnp.float32),
        ] + [pltpu.SemaphoreType.DMA] * (2 * NBUF),
    )(gflat, srcI, dstI)


def _dinv_body(degp_ref, dinv_ref):
    deg = degp_ref[0, :N] + degp_ref[1, :N] + 1.0
    dinv_ref[...] = lax.rsqrt(deg)[:, None]


def _dinv_call(degp):
    return pl.pallas_call(
        _dinv_body,
        grid=(1,),
        in_specs=[pl.BlockSpec((NC, NROW), lambda i: (0, 0))],
        out_specs=pl.BlockSpec((N, 1), lambda i: (0, 0)),
        out_shape=jax.ShapeDtypeStruct((N, 1), jnp.float32),
    )(degp)


def _mm1_body(x_ref, w_ref, dinv_ref, g_ref):
    dinv = dinv_ref[...]
    h = jnp.dot(x_ref[...], w_ref[...], preferred_element_type=jnp.float32)
    g = h * dinv
    for c in range(C):
        g_ref[c] = g[:, c * LANE:(c + 1) * LANE]


def _mid_body(s_ref, dinv_ref, b_ref, w_ref, g_ref):
    dinv = dinv_ref[...]
    acc = jnp.zeros((R, H), jnp.float32)
    for c in range(C):
        a = s_ref[c] * dinv + b_ref[0, c * LANE:(c + 1) * LANE][None, :]
        a = jnp.where(a > 0, a, NEG * a)
        acc = acc + jnp.dot(a, w_ref[c * LANE:(c + 1) * LANE, :],
                            preferred_element_type=jnp.float32)
    g = acc * dinv
    for c in range(C):
        g_ref[c] = g[:, c * LANE:(c + 1) * LANE]


def _fin_body(s_ref, dinv_ref, b_ref, wl_ref, bl_ref, o_ref):
    dinv = dinv_ref[...]
    acc = jnp.broadcast_to(bl_ref[0, :][None, :], (R, F_IN)).astype(jnp.float32)
    for c in range(C):
        a = s_ref[c] * dinv + b_ref[0, c * LANE:(c + 1) * LANE][None, :]
        acc = acc + jnp.dot(a, wl_ref[c * LANE:(c + 1) * LANE, :],
                            preferred_element_type=jnp.float32)
    o_ref[...] = acc


_g_spec = pl.BlockSpec((C, R, LANE), lambda i: (0, i, 0))
_dinv_spec = pl.BlockSpec((R, 1), lambda i: (i, 0))


def _mm1_call(x, W1, dinv):
    return pl.pallas_call(
        _mm1_body,
        grid=(N // R,),
        in_specs=[pl.BlockSpec((R, F_IN), lambda i: (i, 0)),
                  pl.BlockSpec((F_IN, H), lambda i: (0, 0)),
                  _dinv_spec],
        out_specs=_g_spec,
        out_shape=jax.ShapeDtypeStruct((C, N, LANE), jnp.float32),
    )(x, W1, dinv)


def _mid_call(s, dinv, b2d, W):
    return pl.pallas_call(
        _mid_body,
        grid=(N // R,),
        in_specs=[_g_spec,
                  _dinv_spec,
                  pl.BlockSpec((1, H), lambda i: (0, 0)),
                  pl.BlockSpec((H, H), lambda i: (0, 0))],
        out_specs=_g_spec,
        out_shape=jax.ShapeDtypeStruct((C, N, LANE), jnp.float32),
    )(s, dinv, b2d, W)


def _fin_call(s, dinv, b2d, Wl, bl2d):
    return pl.pallas_call(
        _fin_body,
        grid=(N // R,),
        in_specs=[_g_spec,
                  _dinv_spec,
                  pl.BlockSpec((1, H), lambda i: (0, 0)),
                  pl.BlockSpec((H, F_IN), lambda i: (0, 0)),
                  pl.BlockSpec((1, F_IN), lambda i: (0, 0))],
        out_specs=pl.BlockSpec((R, F_IN), lambda i: (i, 0)),
        out_shape=jax.ShapeDtypeStruct((N, F_IN), jnp.float32),
    )(s, dinv, b2d, Wl, bl2d)


def kernel(x, edge_index, W1, b1, W2, b2, W3, b3, Wl, bl):
    src = edge_index[0]
    dst = edge_index[1]
    pad = E_PAD - E
    src_p = jnp.concatenate([src, jnp.zeros((pad,), jnp.int32)])
    dst_p = jnp.concatenate([dst, jnp.full((pad,), N, jnp.int32)])
    dstI = dst_p.reshape(NS * NQ, NGQ, G)
    srcI = (src_p.reshape(1, NS * NQ, NGQ, G)
            + (jnp.arange(C, dtype=jnp.int32) * N)[:, None, None, None]
            ).reshape(C * NS * NQ, NGQ, G)
    degp = _deg_call(dstI)
    dinv = _dinv_call(degp.reshape(NC, NROW))

    g1 = _mm1_call(x, W1, dinv)
    s1 = _scat_call(g1.reshape(C * N, LANE), srcI, dstI)
    g2 = _mid_call(s1, dinv, b1.reshape(1, H), W2)
    s2 = _scat_call(g2.reshape(C * N, LANE), srcI, dstI)
    g3 = _mid_call(s2, dinv, b2.reshape(1, H), W3)
    s3 = _scat_call(g3.reshape(C * N, LANE), srcI, dstI)
    return _fin_call(s3, dinv, b3.reshape(1, H), Wl, bl.reshape(1, F_IN))
